# baseline (speedup 1.0000x reference)
import functools
import jax
import jax.numpy as jnp
from jax import lax
from jax.experimental import pallas as pl
from jax.experimental.pallas import tpu as pltpu

NEG_INF = float("-inf")


def _qkv_kernel(x_ref, w_ref, b_ref, o_ref):
    acc = lax.dot_general(
        x_ref[...], w_ref[...],
        dimension_numbers=(((1,), (1,)), ((), ())),
        preferred_element_type=jnp.float32,
        precision=lax.Precision.DEFAULT,
    )
    hd = o_ref.shape[2]
    for c in range(o_ref.shape[0]):
        o_ref[c] = acc[:, c * hd:(c + 1) * hd] + b_ref[c]


def _attn_proj_kernel(k_top, scale, nheads, q_ref, k_ref, v_ref, wp_ref,
                      b_ref, o_ref, s_ref, ot_ref):
    i = pl.program_id(0)

    @pl.when(i < nheads)
    def _attention_phase():
        nb, w = s_ref.shape
        nfull = q_ref.shape[1]
        nrb = nfull // nb
        nc = w // 128
        k = k_ref[0]
        for rb in range(nrb):
            q = q_ref[0, rb * nb:(rb + 1) * nb, :]
            s = lax.dot_general(
                q, k,
                dimension_numbers=(((1,), (1,)), ((), ())),
                preferred_element_type=jnp.float32,
                precision=lax.Precision.DEFAULT,
            ) * scale
            s_ref[...] = s

            r1 = s_ref[:, 0:128]
            neg = jnp.full((nb, 128), NEG_INF, jnp.float32)
            r2, r3, r4 = neg, neg, neg
            for c in range(1, nc):
                x = s_ref[:, c * 128:(c + 1) * 128]
                t1 = jnp.maximum(r1, x)
                x = jnp.minimum(r1, x)
                r1 = t1
                t2 = jnp.maximum(r2, x)
                x = jnp.minimum(r2, x)
                r2 = t2
                t3 = jnp.maximum(r3, x)
                x = jnp.minimum(r3, x)
                r3 = t3
                r4 = jnp.maximum(r4, x)
            m = jnp.max(r1, axis=1, keepdims=True)

            t = m
            for _ in range(k_top - 1):
                w1 = jnp.maximum(jnp.where(r1 < t, r1, NEG_INF),
                                 jnp.where(r2 < t, r2, NEG_INF))
                w2 = jnp.maximum(jnp.where(r3 < t, r3, NEG_INF),
                                 jnp.where(r4 < t, r4, NEG_INF))
                t = jnp.max(jnp.maximum(w1, w2), axis=1, keepdims=True)

            s = s_ref[...]
            cnt = jnp.sum(jnp.where(s >= t, 1.0, 0.0), axis=1, keepdims=True)
            ok = jnp.all(cnt == jnp.float32(k_top))

            def fallback():
                def body(_, tt):
                    return jnp.max(
                        jnp.where(s_ref[...] < tt, s_ref[...], NEG_INF),
                        axis=1, keepdims=True)
                return lax.fori_loop(0, k_top - 1, body, m)

            t = lax.cond(ok, lambda: t, fallback)
            s = s_ref[...]
            s_ref[...] = jnp.where(s >= t, jnp.exp(s - m), 0.0)
            ot = lax.dot_general(
                v_ref[0], s_ref[...],
                dimension_numbers=(((0,), (1,)), ((), ())),
                preferred_element_type=jnp.float32,
                precision=lax.Precision.DEFAULT,
            )
            denom_t = lax.dot_general(
                jnp.ones((1, w), jnp.float32), s_ref[...],
                dimension_numbers=(((1,), (1,)), ((), ())),
                preferred_element_type=jnp.float32,
                precision=lax.Precision.HIGHEST,
            )
            hd = ot.shape[0]
            ot_ref[pl.ds(i * hd, hd), rb * nb:(rb + 1) * nb] = ot / denom_t

    @pl.when(i >= nheads)
    def _proj_phase():
        acc = lax.dot_general(
            ot_ref[...], wp_ref[...],
            dimension_numbers=(((0,), (1,)), ((), ())),
            preferred_element_type=jnp.float32,
            precision=lax.Precision.DEFAULT,
        )
        o_ref[...] = acc + b_ref[...]


@jax.jit
def kernel(x, Wqkv, bqkv, Wproj, bproj):
    B, N, C = x.shape
    H = 16
    HD = C // H
    K_TOP = 20
    scale = HD ** -0.5
    x2 = x.reshape(N, C)

    nslab = 3 * H
    SL = 4
    qkv = pl.pallas_call(
        _qkv_kernel,
        grid=(nslab // SL,),
        in_specs=[
            pl.BlockSpec((N, C), lambda j: (0, 0)),
            pl.BlockSpec((SL * HD, C), lambda j: (j, 0)),
            pl.BlockSpec((SL, 1, HD), lambda j: (j, 0, 0)),
        ],
        out_specs=pl.BlockSpec((SL, N, HD), lambda j: (j, 0, 0)),
        out_shape=jax.ShapeDtypeStruct((nslab, N, HD), jnp.float32),
    )(x2, Wqkv, bqkv.reshape(nslab, 1, HD))

    PW = 256
    nproj = C // PW
    hmax = H - 1

    attn = functools.partial(_attn_proj_kernel, K_TOP, scale, H)
    out = pl.pallas_call(
        attn,
        grid=(H + nproj,),
        in_specs=[
            pl.BlockSpec((1, N, HD), lambda i: (jnp.minimum(i, hmax), 0, 0)),
            pl.BlockSpec((1, N, HD),
                         lambda i: (jnp.minimum(i, hmax) + 16, 0, 0)),
            pl.BlockSpec((1, N, HD),
                         lambda i: (jnp.minimum(i, hmax) + 32, 0, 0)),
            pl.BlockSpec((PW, C), lambda i: (jnp.maximum(i - 16, 0), 0)),
            pl.BlockSpec((1, PW), lambda i: (0, jnp.maximum(i - 16, 0))),
        ],
        out_specs=pl.BlockSpec((N, PW), lambda i: (0, jnp.maximum(i - 16, 0))),
        out_shape=jax.ShapeDtypeStruct((N, C), jnp.float32),
        scratch_shapes=[pltpu.VMEM((N // 2, N), jnp.float32),
                        pltpu.VMEM((C, N), jnp.float32)],
    )(qkv, qkv, qkv, Wproj, bproj.reshape(1, C))

    return out.reshape(B, N, C)


# speedup vs baseline: 1.5730x; 1.5730x over previous
import functools
import jax
import jax.numpy as jnp
from jax import lax
from jax.experimental import pallas as pl
from jax.experimental.pallas import tpu as pltpu

NEG_INF = float("-inf")


def _qkv_kernel(x_ref, w_ref, b_ref, o_ref):
    acc = lax.dot_general(
        x_ref[...], w_ref[...],
        dimension_numbers=(((1,), (1,)), ((), ())),
        preferred_element_type=jnp.float32,
        precision=lax.Precision.DEFAULT,
    )
    hd = o_ref.shape[2]
    for c in range(o_ref.shape[0]):
        o_ref[c] = acc[:, c * hd:(c + 1) * hd] + b_ref[c]


def _attn_proj_kernel(k_top, scale, nheads, q_ref, k_ref, v_ref, wp_ref,
                      b_ref, o_ref, s_ref, ot_ref):
    i = pl.program_id(0)

    @pl.when(i < nheads)
    def _attention_phase():
        nb, w = s_ref.shape
        nfull = q_ref.shape[1]
        nrb = nfull // nb
        nc = w // 128
        k = k_ref[0]
        for rb in range(nrb):
            q = q_ref[0, rb * nb:(rb + 1) * nb, :]
            s = lax.dot_general(
                q, k,
                dimension_numbers=(((1,), (1,)), ((), ())),
                preferred_element_type=jnp.float32,
                precision=lax.Precision.DEFAULT,
            ) * scale
            s_ref[...] = s

            r1 = s_ref[:, 0:128]
            neg = jnp.full((nb, 128), NEG_INF, jnp.float32)
            r2, r3, r4 = neg, neg, neg
            for c in range(1, nc):
                x = s_ref[:, c * 128:(c + 1) * 128]
                t1 = jnp.maximum(r1, x)
                x = jnp.minimum(r1, x)
                r1 = t1
                t2 = jnp.maximum(r2, x)
                x = jnp.minimum(r2, x)
                r2 = t2
                t3 = jnp.maximum(r3, x)
                x = jnp.minimum(r3, x)
                r3 = t3
                r4 = jnp.maximum(r4, x)
            m = jnp.max(r1, axis=1, keepdims=True)

            t = m
            for _ in range(k_top - 1):
                w1 = jnp.maximum(jnp.where(r1 < t, r1, NEG_INF),
                                 jnp.where(r2 < t, r2, NEG_INF))
                w2 = jnp.maximum(jnp.where(r3 < t, r3, NEG_INF),
                                 jnp.where(r4 < t, r4, NEG_INF))
                t = jnp.max(jnp.maximum(w1, w2), axis=1, keepdims=True)

            s = s_ref[...]
            cnt = jnp.sum(jnp.where(s >= t, 1.0, 0.0), axis=1, keepdims=True)
            ok = jnp.all(cnt == jnp.float32(k_top))

            def fallback():
                def body(_, tt):
                    return jnp.max(
                        jnp.where(s_ref[...] < tt, s_ref[...], NEG_INF),
                        axis=1, keepdims=True)
                return lax.fori_loop(0, k_top - 1, body, m)

            t = lax.cond(ok, lambda: t, fallback)
            s = s_ref[...]
            s_ref[...] = jnp.where(s >= t, jnp.exp(s - m), 0.0)
            o = lax.dot_general(
                s_ref[...], v_ref[0],
                dimension_numbers=(((1,), (0,)), ((), ())),
                preferred_element_type=jnp.float32,
                precision=lax.Precision.DEFAULT,
            )
            denom = lax.dot_general(
                s_ref[...], jnp.ones((1, w), jnp.float32),
                dimension_numbers=(((1,), (1,)), ((), ())),
                preferred_element_type=jnp.float32,
                precision=lax.Precision.HIGHEST,
            )
            hd = o.shape[1]
            ot_ref[rb * nb:(rb + 1) * nb, pl.ds(i * hd, hd)] = o / denom

    @pl.when(i >= nheads)
    def _proj_phase():
        acc = lax.dot_general(
            ot_ref[...], wp_ref[...],
            dimension_numbers=(((1,), (1,)), ((), ())),
            preferred_element_type=jnp.float32,
            precision=lax.Precision.DEFAULT,
        )
        o_ref[...] = acc + b_ref[...]


@jax.jit
def kernel(x, Wqkv, bqkv, Wproj, bproj):
    B, N, C = x.shape
    H = 16
    HD = C // H
    K_TOP = 20
    scale = HD ** -0.5
    x2 = x.reshape(N, C)

    nslab = 3 * H
    SL = 8
    qkv = pl.pallas_call(
        _qkv_kernel,
        grid=(nslab // SL,),
        in_specs=[
            pl.BlockSpec((N, C), lambda j: (0, 0)),
            pl.BlockSpec((SL * HD, C), lambda j: (j, 0)),
            pl.BlockSpec((SL, 1, HD), lambda j: (j, 0, 0)),
        ],
        out_specs=pl.BlockSpec((SL, N, HD), lambda j: (j, 0, 0)),
        out_shape=jax.ShapeDtypeStruct((nslab, N, HD), jnp.float32),
    )(x2, Wqkv, bqkv.reshape(nslab, 1, HD))

    PW = 256
    nproj = C // PW
    hmax = H - 1

    attn = functools.partial(_attn_proj_kernel, K_TOP, scale, H)
    out = pl.pallas_call(
        attn,
        grid=(H + nproj,),
        in_specs=[
            pl.BlockSpec((1, N, HD), lambda i: (jnp.minimum(i, hmax), 0, 0)),
            pl.BlockSpec((1, N, HD),
                         lambda i: (jnp.minimum(i, hmax) + 16, 0, 0)),
            pl.BlockSpec((1, N, HD),
                         lambda i: (jnp.minimum(i, hmax) + 32, 0, 0)),
            pl.BlockSpec((PW, C), lambda i: (jnp.maximum(i - 16, 0), 0)),
            pl.BlockSpec((1, PW), lambda i: (0, jnp.maximum(i - 16, 0))),
        ],
        out_specs=pl.BlockSpec((N, PW), lambda i: (0, jnp.maximum(i - 16, 0))),
        out_shape=jax.ShapeDtypeStruct((N, C), jnp.float32),
        scratch_shapes=[pltpu.VMEM((N // 2, N), jnp.float32),
                        pltpu.VMEM((N, C), jnp.float32)],
    )(qkv, qkv, qkv, Wproj, bproj.reshape(1, C))

    return out.reshape(B, N, C)


# speedup vs baseline: 1.5790x; 1.0038x over previous
import functools
import jax
import jax.numpy as jnp
from jax import lax
from jax.experimental import pallas as pl
from jax.experimental.pallas import tpu as pltpu

NEG_INF = float("-inf")


def _qkv_kernel(x_ref, w_ref, b_ref, o_ref):
    acc = lax.dot_general(
        x_ref[...], w_ref[...],
        dimension_numbers=(((1,), (1,)), ((), ())),
        preferred_element_type=jnp.float32,
        precision=lax.Precision.DEFAULT,
    )
    hd = o_ref.shape[2]
    for c in range(o_ref.shape[0]):
        o_ref[c] = acc[:, c * hd:(c + 1) * hd] + b_ref[c]


def _attn_proj_kernel(k_top, scale, nheads, q_ref, k_ref, v_ref, wp_ref,
                      b_ref, o_ref, s_ref, ot_ref):
    i = pl.program_id(0)

    @pl.when(i < nheads)
    def _attention_phase():
        nb, w = s_ref.shape
        nfull = q_ref.shape[1]
        nrb = nfull // nb
        nc = w // 128
        k = k_ref[0]
        for rb in range(nrb):
            q = q_ref[0, rb * nb:(rb + 1) * nb, :]
            s = lax.dot_general(
                q, k,
                dimension_numbers=(((1,), (1,)), ((), ())),
                preferred_element_type=jnp.float32,
                precision=lax.Precision.DEFAULT,
            ) * scale
            s_ref[...] = s

            r1 = s_ref[:, 0:128]
            neg = jnp.full((nb, 128), NEG_INF, jnp.float32)
            r2, r3, r4 = neg, neg, neg
            for c in range(1, nc):
                x = s_ref[:, c * 128:(c + 1) * 128]
                t1 = jnp.maximum(r1, x)
                x = jnp.minimum(r1, x)
                r1 = t1
                t2 = jnp.maximum(r2, x)
                x = jnp.minimum(r2, x)
                r2 = t2
                t3 = jnp.maximum(r3, x)
                x = jnp.minimum(r3, x)
                r3 = t3
                r4 = jnp.maximum(r4, x)
            m = jnp.max(r1, axis=1, keepdims=True)

            t = m
            for _ in range(k_top - 1):
                w1 = jnp.maximum(jnp.where(r1 < t, r1, NEG_INF),
                                 jnp.where(r2 < t, r2, NEG_INF))
                w2 = jnp.maximum(jnp.where(r3 < t, r3, NEG_INF),
                                 jnp.where(r4 < t, r4, NEG_INF))
                t = jnp.max(jnp.maximum(w1, w2), axis=1, keepdims=True)

            s = s_ref[...]
            cnt = lax.dot_general(
                jnp.where(s >= t, 1.0, 0.0),
                jnp.ones((1, w), jnp.float32),
                dimension_numbers=(((1,), (1,)), ((), ())),
                preferred_element_type=jnp.float32,
                precision=lax.Precision.DEFAULT,
            )
            ok = jnp.all(cnt == jnp.float32(k_top))

            def fallback():
                def body(_, tt):
                    return jnp.max(
                        jnp.where(s_ref[...] < tt, s_ref[...], NEG_INF),
                        axis=1, keepdims=True)
                return lax.fori_loop(0, k_top - 1, body, m)

            t = lax.cond(ok, lambda: t, fallback)
            s = s_ref[...]
            s_ref[...] = jnp.where(s >= t, jnp.exp(s - m), 0.0)
            o = lax.dot_general(
                s_ref[...], v_ref[0],
                dimension_numbers=(((1,), (0,)), ((), ())),
                preferred_element_type=jnp.float32,
                precision=lax.Precision.DEFAULT,
            )
            denom = lax.dot_general(
                s_ref[...], jnp.ones((1, w), jnp.float32),
                dimension_numbers=(((1,), (1,)), ((), ())),
                preferred_element_type=jnp.float32,
                precision=lax.Precision.HIGHEST,
            )
            hd = o.shape[1]
            ot_ref[rb * nb:(rb + 1) * nb, pl.ds(i * hd, hd)] = o / denom

    @pl.when(i >= nheads)
    def _proj_phase():
        acc = lax.dot_general(
            ot_ref[...], wp_ref[...],
            dimension_numbers=(((1,), (1,)), ((), ())),
            preferred_element_type=jnp.float32,
            precision=lax.Precision.DEFAULT,
        )
        o_ref[...] = acc + b_ref[...]


@jax.jit
def kernel(x, Wqkv, bqkv, Wproj, bproj):
    B, N, C = x.shape
    H = 16
    HD = C // H
    K_TOP = 20
    scale = HD ** -0.5
    x2 = x.reshape(N, C)

    nslab = 3 * H
    SL = 8
    qkv = pl.pallas_call(
        _qkv_kernel,
        grid=(nslab // SL,),
        in_specs=[
            pl.BlockSpec((N, C), lambda j: (0, 0)),
            pl.BlockSpec((SL * HD, C), lambda j: (j, 0)),
            pl.BlockSpec((SL, 1, HD), lambda j: (j, 0, 0)),
        ],
        out_specs=pl.BlockSpec((SL, N, HD), lambda j: (j, 0, 0)),
        out_shape=jax.ShapeDtypeStruct((nslab, N, HD), jnp.float32),
    )(x2, Wqkv, bqkv.reshape(nslab, 1, HD))

    PW = 512
    nproj = C // PW
    hmax = H - 1

    attn = functools.partial(_attn_proj_kernel, K_TOP, scale, H)
    out = pl.pallas_call(
        attn,
        grid=(H + nproj,),
        in_specs=[
            pl.BlockSpec((1, N, HD), lambda i: (jnp.minimum(i, hmax), 0, 0)),
            pl.BlockSpec((1, N, HD),
                         lambda i: (jnp.minimum(i, hmax) + 16, 0, 0)),
            pl.BlockSpec((1, N, HD),
                         lambda i: (jnp.minimum(i, hmax) + 32, 0, 0)),
            pl.BlockSpec((PW, C), lambda i: (jnp.maximum(i - 16, 0), 0)),
            pl.BlockSpec((1, PW), lambda i: (0, jnp.maximum(i - 16, 0))),
        ],
        out_specs=pl.BlockSpec((N, PW), lambda i: (0, jnp.maximum(i - 16, 0))),
        out_shape=jax.ShapeDtypeStruct((N, C), jnp.float32),
        scratch_shapes=[pltpu.VMEM((N // 2, N), jnp.float32),
                        pltpu.VMEM((N, C), jnp.float32)],
    )(qkv, qkv, qkv, Wproj, bproj.reshape(1, C))

    return out.reshape(B, N, C)
